# baseline (speedup 1.0000x reference)
import jax
import jax.numpy as jnp
from jax import lax
from jax.experimental import pallas as pl
from jax.experimental.pallas import tpu as pltpu
from jax.experimental.pallas import tpu_sc as plsc

NUM_EMB = 100000
DIM = 64
BATCH = 4096
HIST = 50

_NC = 2
_NS = 16
_NW = _NC * _NS

_B = BATCH * HIST
_BPW = _B // _NW
_CH = 128
_NCH = HIST
_NBUF = 5
_L = 16
_DB = DIM // 8


def _gather_body(idx_hbm, table_hbm, mask_hbm, out_hbm,
                 idx_v, b_0, b_1, b_2, b_3, b_4,
                 t_0, t_1, t_2, t_3, t_4, mbufs, gsem, msem, wsem):
    bufs = [b_0, b_1, b_2, b_3, b_4]
    tbufs = [t_0, t_1, t_2, t_3, t_4]
    wid = lax.axis_index("s") * _NC + lax.axis_index("c")
    base = wid * _BPW

    pltpu.sync_copy(idx_hbm.at[pl.ds(base, _BPW)], idx_v)

    def idx_slice(j):
        return idx_v.at[pl.ds(j * _CH, _CH)]

    def gather(j, b):
        pltpu.async_copy(table_hbm.at[idx_slice(j)], bufs[b], gsem.at[b])
        pltpu.async_copy(mask_hbm.at[idx_slice(j)], mbufs.at[b], msem.at[b])

    def wait_gather(j, b):
        pltpu.make_async_copy(
            table_hbm.at[idx_slice(j)], bufs[b], gsem.at[b]
        ).wait()
        pltpu.make_async_copy(
            mask_hbm.at[idx_slice(j)], mbufs.at[b], msem.at[b]
        ).wait()

    def write(j, b):
        pltpu.async_copy(
            tbufs[b], out_hbm.at[j, :, wid, :, :], wsem.at[b]
        )

    def wait_write(j, b):
        pltpu.make_async_copy(
            tbufs[b], out_hbm.at[j, :, wid, :, :], wsem.at[b]
        ).wait()

    row_iota = lax.iota(jnp.int32, _L)

    def mul_transpose(b):
        def grp(g, _):
            rows = row_iota + g * _L
            m16 = mbufs[b, pl.ds(g * _L, _L)]

            for d in range(DIM):
                v = plsc.load_gather(
                    bufs[b], [rows, jnp.full((_L,), d, jnp.int32)]
                )
                tbufs[b][d // 8, d % 8, pl.ds(g * _L, _L)] = v * m16
            return _

        lax.fori_loop(0, _CH // _L, grp, 0, unroll=False)

    for b in range(_NBUF):
        gather(b, b)

    def group(i, _):
        g = i * _NBUF
        for b in range(_NBUF):
            j = g + b
            wait_gather(j, b)
            mul_transpose(b)
            write(j, b)
            wait_write(j, b)
            gather(j + _NBUF, b)
        return _

    lax.fori_loop(0, _NCH // _NBUF - 1, group, 0, unroll=False)

    g = _NCH - _NBUF
    for b in range(_NBUF):
        j = g + b
        wait_gather(j, b)
        mul_transpose(b)
        write(j, b)
    for b in range(_NBUF):
        wait_write(g + b, b)


@jax.jit
def _gather(idx, table, mask1):
    mesh = plsc.VectorSubcoreMesh(core_axis_name="c", subcore_axis_name="s")
    return pl.kernel(
        _gather_body,
        mesh=mesh,
        out_type=jax.ShapeDtypeStruct(
            (HIST, _DB, _NW, 8, 128), jnp.float32
        ),
        scratch_types=[
            pltpu.VMEM((_BPW,), jnp.int32),
        ] + [pltpu.VMEM((_CH, DIM), jnp.float32) for _ in range(_NBUF)]
        + [pltpu.VMEM((_DB, 8, 128), jnp.float32) for _ in range(_NBUF)]
        + [
            pltpu.VMEM((_NBUF, _CH), jnp.float32),
            pltpu.SemaphoreType.DMA((_NBUF,)),
            pltpu.SemaphoreType.DMA((_NBUF,)),
            pltpu.SemaphoreType.DMA((_NBUF,)),
        ],
        compiler_params=pltpu.CompilerParams(
            use_tc_tiling_on_sc=False, needs_layout_passes=False
        ),
    )(idx, table, mask1)


def kernel(words, table, mask):
    idx = words.reshape(_NW, _CH, HIST).transpose(0, 2, 1).reshape(_B)
    mask1 = mask.reshape(NUM_EMB)
    out5 = _gather(idx, table, mask1)
    return out5.transpose(2, 4, 0, 1, 3).reshape(BATCH, HIST, DIM)


# speedup vs baseline: 2.0158x; 2.0158x over previous
import jax
import jax.numpy as jnp
from jax import lax
from jax.experimental import pallas as pl
from jax.experimental.pallas import tpu as pltpu
from jax.experimental.pallas import tpu_sc as plsc

NUM_EMB = 100000
DIM = 64
BATCH = 4096
HIST = 50

_NC = 2
_NS = 16
_NW = _NC * _NS

_B = BATCH * HIST
_BPW = _B // _NW
_CH = 128
_NCH = HIST
_NBUF = 5
_L = 16
_DB = DIM // 8
_PAD = 129


def _gather_body(idx_hbm, table_hbm, mask_hbm, out_hbm,
                 idx_v, b_0, b_1, b_2, b_3, b_4,
                 t_0, t_1, t_2, t_3, t_4,
                 m_0, m_1, m_2, m_3, m_4, gsem, msem, wsem):
    bufs = [b_0, b_1, b_2, b_3, b_4]
    tbufs = [t_0, t_1, t_2, t_3, t_4]
    mbufs = [m_0, m_1, m_2, m_3, m_4]
    wid = lax.axis_index("s") * _NC + lax.axis_index("c")
    base = wid * _BPW

    pltpu.sync_copy(idx_hbm.at[pl.ds(base, _BPW)], idx_v)

    def idx_slice(j):
        return idx_v.at[pl.ds(j * _CH, _CH)]

    def gather(j, b):
        pltpu.async_copy(table_hbm.at[idx_slice(j)], bufs[b], gsem.at[b])
        pltpu.async_copy(mask_hbm.at[idx_slice(j)], mbufs[b], msem.at[b])

    def wait_gather(j, b):
        pltpu.make_async_copy(
            table_hbm.at[idx_slice(j)], bufs[b], gsem.at[b]
        ).wait()
        pltpu.make_async_copy(
            mask_hbm.at[idx_slice(j)], mbufs[b], msem.at[b]
        ).wait()

    def write(j, b):
        pltpu.async_copy(
            tbufs[b].at[:, :, pl.ds(0, 128)],
            out_hbm.at[j, :, wid, :, :], wsem.at[b]
        )

    def wait_write(j, b):
        pltpu.make_async_copy(
            tbufs[b].at[:, :, pl.ds(0, 128)],
            out_hbm.at[j, :, wid, :, :], wsem.at[b]
        ).wait()

    lane_iota = lax.iota(jnp.int32, _L)
    d_blk = [(lane_iota + k * _L) // 8 for k in range(DIM // _L)]
    d_sub = [(lane_iota + k * _L) % 8 for k in range(DIM // _L)]

    def mul_transpose(b):
        def row(r, _):
            r16 = jnp.full((_L,), 0, jnp.int32) + r
            m = plsc.load_gather(mbufs[b], [r16])
            for k in range(DIM // _L):
                v = bufs[b][r, pl.ds(k * _L, _L)] * m
                plsc.store_scatter(tbufs[b], [d_blk[k], d_sub[k], r16], v)
            return _

        lax.fori_loop(0, _CH, row, 0, unroll=False)

    for b in range(_NBUF):
        gather(b, b)

    def group(i, _):
        g = i * _NBUF
        for b in range(_NBUF):
            j = g + b
            wait_gather(j, b)
            mul_transpose(b)
            write(j, b)
            wait_write(j, b)
            gather(j + _NBUF, b)
        return _

    lax.fori_loop(0, _NCH // _NBUF - 1, group, 0, unroll=False)

    g = _NCH - _NBUF
    for b in range(_NBUF):
        j = g + b
        wait_gather(j, b)
        mul_transpose(b)
        write(j, b)
    for b in range(_NBUF):
        wait_write(g + b, b)


@jax.jit
def _gather(idx, table, mask1):
    mesh = plsc.VectorSubcoreMesh(core_axis_name="c", subcore_axis_name="s")
    return pl.kernel(
        _gather_body,
        mesh=mesh,
        out_type=jax.ShapeDtypeStruct(
            (HIST, _DB, _NW, 8, 128), jnp.float32
        ),
        scratch_types=[
            pltpu.VMEM((_BPW,), jnp.int32),
        ] + [pltpu.VMEM((_CH, DIM), jnp.float32) for _ in range(_NBUF)]
        + [pltpu.VMEM((_DB, 8, _PAD), jnp.float32) for _ in range(_NBUF)]
        + [pltpu.VMEM((_CH,), jnp.float32) for _ in range(_NBUF)]
        + [
            pltpu.SemaphoreType.DMA((_NBUF,)),
            pltpu.SemaphoreType.DMA((_NBUF,)),
            pltpu.SemaphoreType.DMA((_NBUF,)),
        ],
        compiler_params=pltpu.CompilerParams(
            use_tc_tiling_on_sc=False, needs_layout_passes=False
        ),
    )(idx, table, mask1)


def kernel(words, table, mask):
    idx = words.reshape(_NW, _CH, HIST).transpose(0, 2, 1).reshape(_B)
    mask1 = mask.reshape(NUM_EMB)
    out5 = _gather(idx, table, mask1)
    return out5.transpose(2, 4, 0, 1, 3).reshape(BATCH, HIST, DIM)


# speedup vs baseline: 3.3963x; 1.6849x over previous
import jax
import jax.numpy as jnp
from jax import lax
from jax.experimental import pallas as pl
from jax.experimental.pallas import tpu as pltpu
from jax.experimental.pallas import tpu_sc as plsc

NUM_EMB = 100000
DIM = 64
BATCH = 4096
HIST = 50

_NC = 2
_NS = 16
_NW = _NC * _NS

_B = BATCH * HIST
_BPW = _B // _NW
_CH = 128
_NCH = HIST
_NBUF = 5
_L = 16
_DB = DIM // 8
_PAD = 129


def _gather_body(idx_hbm, table_hbm, mask_hbm, out_hbm,
                 idx_v, b_0, b_1, b_2, b_3, b_4,
                 t_0, t_1, t_2, t_3, t_4,
                 m_0, m_1, m_2, m_3, m_4, gsem, msem, wsem):
    bufs = [b_0, b_1, b_2, b_3, b_4]
    tbufs = [t_0, t_1, t_2, t_3, t_4]
    mbufs = [m_0, m_1, m_2, m_3, m_4]
    wid = lax.axis_index("s") * _NC + lax.axis_index("c")
    base = wid * _BPW

    pltpu.sync_copy(idx_hbm.at[pl.ds(base, _BPW)], idx_v)

    def idx_slice(j):
        return idx_v.at[pl.ds(j * _CH, _CH)]

    def gather(j, b):
        pltpu.async_copy(table_hbm.at[idx_slice(j)], bufs[b], gsem.at[b])
        pltpu.async_copy(mask_hbm.at[idx_slice(j)], mbufs[b], msem.at[b])

    def wait_gather(j, b):
        pltpu.make_async_copy(
            table_hbm.at[idx_slice(j)], bufs[b], gsem.at[b]
        ).wait()
        pltpu.make_async_copy(
            mask_hbm.at[idx_slice(j)], mbufs[b], msem.at[b]
        ).wait()

    def write(j, b):
        pltpu.async_copy(
            tbufs[b].at[:, :, pl.ds(0, 128)],
            out_hbm.at[j, :, wid, :, :], wsem.at[b]
        )

    def wait_write(j, b):
        pltpu.make_async_copy(
            tbufs[b].at[:, :, pl.ds(0, 128)],
            out_hbm.at[j, :, wid, :, :], wsem.at[b]
        ).wait()

    lane_iota = lax.iota(jnp.int32, _L)
    d_blk = [(lane_iota + k * _L) // 8 for k in range(DIM // _L)]
    d_sub = [(lane_iota + k * _L) % 8 for k in range(DIM // _L)]

    def mul_transpose(b):
        @plsc.parallel_loop(0, _CH, 1, unroll=8)
        def _row(r):
            r16 = jnp.full((_L,), 0, jnp.int32) + r
            m = plsc.load_gather(mbufs[b], [r16])
            for k in range(DIM // _L):
                v = bufs[b][r, pl.ds(k * _L, _L)] * m
                plsc.store_scatter(tbufs[b], [d_blk[k], d_sub[k], r16], v)

    for b in range(_NBUF):
        gather(b, b)

    def group(i, _):
        g = i * _NBUF
        for b in range(_NBUF):
            j = g + b
            wait_gather(j, b)
            mul_transpose(b)
            write(j, b)
            wait_write(j, b)
            gather(j + _NBUF, b)
        return _

    lax.fori_loop(0, _NCH // _NBUF - 1, group, 0, unroll=False)

    g = _NCH - _NBUF
    for b in range(_NBUF):
        j = g + b
        wait_gather(j, b)
        mul_transpose(b)
        write(j, b)
    for b in range(_NBUF):
        wait_write(g + b, b)


@jax.jit
def _gather(idx, table, mask1):
    mesh = plsc.VectorSubcoreMesh(core_axis_name="c", subcore_axis_name="s")
    return pl.kernel(
        _gather_body,
        mesh=mesh,
        out_type=jax.ShapeDtypeStruct(
            (HIST, _DB, _NW, 8, 128), jnp.float32
        ),
        scratch_types=[
            pltpu.VMEM((_BPW,), jnp.int32),
        ] + [pltpu.VMEM((_CH, DIM), jnp.float32) for _ in range(_NBUF)]
        + [pltpu.VMEM((_DB, 8, _PAD), jnp.float32) for _ in range(_NBUF)]
        + [pltpu.VMEM((_CH,), jnp.float32) for _ in range(_NBUF)]
        + [
            pltpu.SemaphoreType.DMA((_NBUF,)),
            pltpu.SemaphoreType.DMA((_NBUF,)),
            pltpu.SemaphoreType.DMA((_NBUF,)),
        ],
        compiler_params=pltpu.CompilerParams(
            use_tc_tiling_on_sc=False, needs_layout_passes=False
        ),
    )(idx, table, mask1)


def kernel(words, table, mask):
    idx = words.reshape(_NW, _CH, HIST).transpose(0, 2, 1).reshape(_B)
    mask1 = mask.reshape(NUM_EMB)
    out5 = _gather(idx, table, mask1)
    return out5.transpose(2, 4, 0, 1, 3).reshape(BATCH, HIST, DIM)


# speedup vs baseline: 3.3982x; 1.0005x over previous
import jax
import jax.numpy as jnp
from jax import lax
from jax.experimental import pallas as pl
from jax.experimental.pallas import tpu as pltpu
from jax.experimental.pallas import tpu_sc as plsc

NUM_EMB = 100000
DIM = 64
BATCH = 4096
HIST = 50

_NC = 2
_NS = 16
_NW = _NC * _NS

_B = BATCH * HIST
_BPW = _B // _NW
_CH = 128
_NCH = HIST
_NBUF = 5
_L = 16
_DB = DIM // 8
_PAD = 129


def _gather_body(idx_hbm, table_hbm, mask_hbm, out_hbm,
                 idx_v, idx_t, b_0, b_1, b_2, b_3, b_4,
                 t_0, t_1, t_2, t_3, t_4,
                 m_0, m_1, m_2, m_3, m_4, gsem, msem, wsem):
    bufs = [b_0, b_1, b_2, b_3, b_4]
    tbufs = [t_0, t_1, t_2, t_3, t_4]
    mbufs = [m_0, m_1, m_2, m_3, m_4]
    wid = lax.axis_index("s") * _NC + lax.axis_index("c")
    base = wid * _BPW

    pltpu.sync_copy(idx_hbm.at[pl.ds(base, _BPW)], idx_v)

    hist_iota = lax.iota(jnp.int32, _L) * HIST

    @plsc.parallel_loop(0, _NCH * (_CH // _L), 1, unroll=8)
    def _tr(t):
        h = t // (_CH // _L)
        g = t % (_CH // _L)
        v = plsc.load_gather(idx_v, [hist_iota + (g * _L * HIST + h)])
        idx_t[pl.ds(h * _CH + g * _L, _L)] = v

    def idx_slice(j):
        return idx_t.at[pl.ds(j * _CH, _CH)]

    def gather(j, b):
        pltpu.async_copy(table_hbm.at[idx_slice(j)], bufs[b], gsem.at[b])
        pltpu.async_copy(mask_hbm.at[idx_slice(j)], mbufs[b], msem.at[b])

    def wait_gather(j, b):
        pltpu.make_async_copy(
            table_hbm.at[idx_slice(j)], bufs[b], gsem.at[b]
        ).wait()
        pltpu.make_async_copy(
            mask_hbm.at[idx_slice(j)], mbufs[b], msem.at[b]
        ).wait()

    def write(j, b):
        pltpu.async_copy(
            tbufs[b].at[:, :, pl.ds(0, 128)],
            out_hbm.at[j, :, wid, :, :], wsem.at[b]
        )

    def wait_write(j, b):
        pltpu.make_async_copy(
            tbufs[b].at[:, :, pl.ds(0, 128)],
            out_hbm.at[j, :, wid, :, :], wsem.at[b]
        ).wait()

    lane_iota = lax.iota(jnp.int32, _L)
    d_blk = [(lane_iota + k * _L) // 8 for k in range(DIM // _L)]
    d_sub = [(lane_iota + k * _L) % 8 for k in range(DIM // _L)]

    def mul_transpose(b):
        @plsc.parallel_loop(0, _CH, 1, unroll=8)
        def _row(r):
            r16 = jnp.full((_L,), 0, jnp.int32) + r
            m = plsc.load_gather(mbufs[b], [r16])
            for k in range(DIM // _L):
                v = bufs[b][r, pl.ds(k * _L, _L)] * m
                plsc.store_scatter(tbufs[b], [d_blk[k], d_sub[k], r16], v)

    for b in range(_NBUF):
        gather(b, b)

    def group(i, _):
        g = i * _NBUF
        for b in range(_NBUF):
            j = g + b
            wait_gather(j, b)
            mul_transpose(b)
            write(j, b)
            wait_write(j, b)
            gather(j + _NBUF, b)
        return _

    lax.fori_loop(0, _NCH // _NBUF - 1, group, 0, unroll=False)

    g = _NCH - _NBUF
    for b in range(_NBUF):
        j = g + b
        wait_gather(j, b)
        mul_transpose(b)
        write(j, b)
    for b in range(_NBUF):
        wait_write(g + b, b)


@jax.jit
def _gather(idx, table, mask1):
    mesh = plsc.VectorSubcoreMesh(core_axis_name="c", subcore_axis_name="s")
    return pl.kernel(
        _gather_body,
        mesh=mesh,
        out_type=jax.ShapeDtypeStruct(
            (HIST, _DB, _NW, 8, 128), jnp.float32
        ),
        scratch_types=[
            pltpu.VMEM((_BPW,), jnp.int32),
            pltpu.VMEM((_BPW,), jnp.int32),
        ] + [pltpu.VMEM((_CH, DIM), jnp.float32) for _ in range(_NBUF)]
        + [pltpu.VMEM((_DB, 8, _PAD), jnp.float32) for _ in range(_NBUF)]
        + [pltpu.VMEM((_CH,), jnp.float32) for _ in range(_NBUF)]
        + [
            pltpu.SemaphoreType.DMA((_NBUF,)),
            pltpu.SemaphoreType.DMA((_NBUF,)),
            pltpu.SemaphoreType.DMA((_NBUF,)),
        ],
        compiler_params=pltpu.CompilerParams(
            use_tc_tiling_on_sc=False, needs_layout_passes=False
        ),
    )(idx, table, mask1)


def kernel(words, table, mask):
    idx = words.reshape(_B)
    mask1 = mask.reshape(NUM_EMB)
    out5 = _gather(idx, table, mask1)
    return out5.transpose(2, 4, 0, 1, 3).reshape(BATCH, HIST, DIM)
